# Initial kernel scaffold; baseline (speedup 1.0000x reference)
import jax
import jax.numpy as jnp
from jax.experimental import pallas as pl


def kernel(x, h, edge_index, W_ih, W_hh, b_ih, b_hh):
    raise NotImplementedError("write your pallas kernel here")


# speedup vs baseline: 3.2764x; 3.2764x over previous
import functools

import jax
import jax.numpy as jnp
from jax import lax
from jax.experimental import pallas as pl
from jax.experimental.pallas import tpu as pltpu
from jax.experimental.pallas import tpu_sc as plsc

N_NODES = 10000
N_EDGES = 320000
HDIM = 128

NC = 2
NS = 16
NW = NC * NS
CHUNK = 128
EDGES_PER_W = 10240
CHUNKS_PER_W = EDGES_PER_W // CHUNK
E_PAD = NW * EDGES_PER_W
ACC_ROWS = 10112
ZCHUNK = ACC_ROWS // NS


def _sc_segment_sum(h, src2d, dst2d, zeros):
    mesh = plsc.VectorSubcoreMesh(core_axis_name="c", subcore_axis_name="s")

    @functools.partial(
        pl.kernel,
        out_type=jax.ShapeDtypeStruct((NC, ACC_ROWS, HDIM), jnp.float32),
        mesh=mesh,
        scratch_types=[
            pltpu.VMEM((CHUNKS_PER_W, CHUNK), jnp.int32),
            pltpu.VMEM((CHUNKS_PER_W, CHUNK), jnp.int32),
            pltpu.VMEM((CHUNK, HDIM), jnp.float32),
            pltpu.VMEM_SHARED((ACC_ROWS, HDIM), jnp.float32),
            pltpu.SemaphoreType.DMA,
        ],
    )
    def sc_kernel(h_hbm, src_hbm, dst_hbm, z_hbm, out_hbm, src_v, dst_v, buf, acc, sem):
        cid = lax.axis_index("c")
        sid = lax.axis_index("s")
        wid = cid * NS + sid

        pltpu.sync_copy(z_hbm.at[pl.ds(sid * ZCHUNK, ZCHUNK)],
                        acc.at[pl.ds(sid * ZCHUNK, ZCHUNK)])

        pltpu.sync_copy(src_hbm.at[pl.ds(wid * CHUNKS_PER_W, CHUNKS_PER_W)], src_v)
        pltpu.sync_copy(dst_hbm.at[pl.ds(wid * CHUNKS_PER_W, CHUNKS_PER_W)], dst_v)
        plsc.subcore_barrier()

        def body(j, carry):
            pltpu.async_copy(h_hbm.at[src_v.at[j]], buf, sem).wait()
            pltpu.sync_copy(buf, acc.at[dst_v.at[j]], add=True)
            return carry

        lax.fori_loop(0, CHUNKS_PER_W, body, 0, unroll=False)

        plsc.subcore_barrier()
        pltpu.sync_copy(acc.at[pl.ds(sid * ZCHUNK, ZCHUNK)],
                        out_hbm.at[cid, pl.ds(sid * ZCHUNK, ZCHUNK)])

    return sc_kernel(h, src2d, dst2d, zeros)


def _gru_body(x_ref, p_ref, wihT_ref, whhT_ref, bih_ref, bhh_ref, o_ref):
    red = p_ref[0] + p_ref[1]
    gi = jnp.dot(x_ref[...], wihT_ref[...], preferred_element_type=jnp.float32)
    gi = gi + bih_ref[...]
    gh = jnp.dot(red, whhT_ref[...], preferred_element_type=jnp.float32)
    gh = gh + bhh_ref[...]
    H = HDIM
    r = jax.nn.sigmoid(gi[:, :H] + gh[:, :H])
    z = jax.nn.sigmoid(gi[:, H:2 * H] + gh[:, H:2 * H])
    n = jnp.tanh(gi[:, 2 * H:] + r * gh[:, 2 * H:])
    o_ref[...] = (1.0 - z) * n + z * red


def _gru_apply(x, partials, wihT, whhT, bih2, bhh2):
    BLK = 1000
    grid = N_NODES // BLK
    return pl.pallas_call(
        _gru_body,
        grid=(grid,),
        in_specs=[
            pl.BlockSpec((BLK, HDIM), lambda i: (i, 0)),
            pl.BlockSpec((NC, BLK, HDIM), lambda i: (0, i, 0)),
            pl.BlockSpec((HDIM, 3 * HDIM), lambda i: (0, 0)),
            pl.BlockSpec((HDIM, 3 * HDIM), lambda i: (0, 0)),
            pl.BlockSpec((1, 3 * HDIM), lambda i: (0, 0)),
            pl.BlockSpec((1, 3 * HDIM), lambda i: (0, 0)),
        ],
        out_specs=pl.BlockSpec((BLK, HDIM), lambda i: (i, 0)),
        out_shape=jax.ShapeDtypeStruct((N_NODES, HDIM), jnp.float32),
    )(x, partials, wihT, whhT, bih2, bhh2)


def kernel(x, h, edge_index, W_ih, W_hh, b_ih, b_hh):
    src = edge_index[0].astype(jnp.int32)
    dst = edge_index[1].astype(jnp.int32)
    pad = E_PAD - N_EDGES
    src_p = jnp.concatenate([src, jnp.zeros((pad,), jnp.int32)]).reshape(
        NW * CHUNKS_PER_W, CHUNK)
    dst_p = jnp.concatenate([dst, jnp.full((pad,), N_NODES, jnp.int32)]).reshape(
        NW * CHUNKS_PER_W, CHUNK)
    zeros = jnp.zeros((ACC_ROWS, HDIM), jnp.float32)

    partials = _sc_segment_sum(h, src_p, dst_p, zeros)

    return _gru_apply(x, partials, W_ih.T, W_hh.T,
                      b_ih.reshape(1, -1), b_hh.reshape(1, -1))


# speedup vs baseline: 3.6092x; 1.1016x over previous
import functools

import jax
import jax.numpy as jnp
from jax import lax
from jax.experimental import pallas as pl
from jax.experimental.pallas import tpu as pltpu
from jax.experimental.pallas import tpu_sc as plsc

N_NODES = 10000
N_EDGES = 320000
HDIM = 128

NC = 2
NS = 16
NW = NC * NS
CHUNK = 128
EDGES_PER_W = 10240
CHUNKS_PER_W = EDGES_PER_W // CHUNK
E_PAD = NW * EDGES_PER_W
ACC_ROWS = 10112
ZCHUNK = ACC_ROWS // NS


def _sc_segment_sum(h, src2d, dst2d, zeros):
    mesh = plsc.VectorSubcoreMesh(core_axis_name="c", subcore_axis_name="s")

    K = 2
    HALF = CHUNKS_PER_W // 2
    ROUNDS = HALF // K

    @functools.partial(
        pl.kernel,
        out_type=jax.ShapeDtypeStruct((NC, ACC_ROWS, HDIM), jnp.float32),
        mesh=mesh,
        scratch_types=(
            [pltpu.VMEM((HALF, CHUNK), jnp.int32),
             pltpu.VMEM((HALF, CHUNK), jnp.int32)]
            + [pltpu.VMEM((CHUNK, HDIM), jnp.float32) for _ in range(K)]
            + [pltpu.VMEM_SHARED((ACC_ROWS, HDIM), jnp.float32)]
            + [pltpu.SemaphoreType.DMA for _ in range(2 * K)]
        ),
    )
    def sc_kernel(h_hbm, src_hbm, dst_hbm, z_hbm, out_hbm, src_v, dst_v, *rest):
        bufs = rest[:K]
        acc = rest[K]
        gsem = rest[K + 1:2 * K + 1]
        ssem = rest[2 * K + 1:]
        cid = lax.axis_index("c")
        sid = lax.axis_index("s")
        wid = cid * NS + sid

        pltpu.sync_copy(z_hbm.at[pl.ds(sid * ZCHUNK, ZCHUNK)],
                        acc.at[pl.ds(sid * ZCHUNK, ZCHUNK)])

        for half in range(2):
            base = wid * CHUNKS_PER_W + half * HALF
            pltpu.sync_copy(src_hbm.at[pl.ds(base, HALF)], src_v)
            pltpu.sync_copy(dst_hbm.at[pl.ds(base, HALF)], dst_v)
            if half == 0:
                plsc.subcore_barrier()

            for b in range(K):
                pltpu.async_copy(h_hbm.at[src_v.at[b]], bufs[b], gsem[b])

            def round_body(t, carry):
                for b in range(K):
                    j = t * K + b
                    pltpu.make_async_copy(
                        h_hbm.at[src_v.at[j]], bufs[b], gsem[b]).wait()
                    pltpu.async_copy(bufs[b], acc.at[dst_v.at[j]], ssem[b],
                                     add=True)

                    @pl.when(j + K < HALF)
                    def _():
                        pltpu.make_async_copy(
                            bufs[b], acc.at[dst_v.at[j]], ssem[b]).wait()
                        pltpu.async_copy(h_hbm.at[src_v.at[j + K]], bufs[b],
                                         gsem[b])
                return carry

            lax.fori_loop(0, ROUNDS, round_body, 0, unroll=False)
            for b in range(K):
                j = (ROUNDS - 1) * K + b
                pltpu.make_async_copy(bufs[b], acc.at[dst_v.at[j]],
                                      ssem[b]).wait()

        plsc.subcore_barrier()
        pltpu.sync_copy(acc.at[pl.ds(sid * ZCHUNK, ZCHUNK)],
                        out_hbm.at[cid, pl.ds(sid * ZCHUNK, ZCHUNK)])

    return sc_kernel(h, src2d, dst2d, zeros)


def _gru_body(x_ref, p_ref, wihT_ref, whhT_ref, bih_ref, bhh_ref, o_ref):
    red = p_ref[0] + p_ref[1]
    gi = jnp.dot(x_ref[...], wihT_ref[...], preferred_element_type=jnp.float32)
    gi = gi + bih_ref[...]
    gh = jnp.dot(red, whhT_ref[...], preferred_element_type=jnp.float32)
    gh = gh + bhh_ref[...]
    H = HDIM
    r = jax.nn.sigmoid(gi[:, :H] + gh[:, :H])
    z = jax.nn.sigmoid(gi[:, H:2 * H] + gh[:, H:2 * H])
    n = jnp.tanh(gi[:, 2 * H:] + r * gh[:, 2 * H:])
    o_ref[...] = (1.0 - z) * n + z * red


def _gru_apply(x, partials, wihT, whhT, bih2, bhh2):
    BLK = 1000
    grid = N_NODES // BLK
    return pl.pallas_call(
        _gru_body,
        grid=(grid,),
        in_specs=[
            pl.BlockSpec((BLK, HDIM), lambda i: (i, 0)),
            pl.BlockSpec((NC, BLK, HDIM), lambda i: (0, i, 0)),
            pl.BlockSpec((HDIM, 3 * HDIM), lambda i: (0, 0)),
            pl.BlockSpec((HDIM, 3 * HDIM), lambda i: (0, 0)),
            pl.BlockSpec((1, 3 * HDIM), lambda i: (0, 0)),
            pl.BlockSpec((1, 3 * HDIM), lambda i: (0, 0)),
        ],
        out_specs=pl.BlockSpec((BLK, HDIM), lambda i: (i, 0)),
        out_shape=jax.ShapeDtypeStruct((N_NODES, HDIM), jnp.float32),
    )(x, partials, wihT, whhT, bih2, bhh2)


def kernel(x, h, edge_index, W_ih, W_hh, b_ih, b_hh):
    src = edge_index[0].astype(jnp.int32)
    dst = edge_index[1].astype(jnp.int32)
    pad = E_PAD - N_EDGES
    src_p = jnp.concatenate([src, jnp.zeros((pad,), jnp.int32)]).reshape(
        NW * CHUNKS_PER_W, CHUNK)
    dst_p = jnp.concatenate([dst, jnp.full((pad,), N_NODES, jnp.int32)]).reshape(
        NW * CHUNKS_PER_W, CHUNK)
    zeros = jnp.zeros((ACC_ROWS, HDIM), jnp.float32)

    partials = _sc_segment_sum(h, src_p, dst_p, zeros)

    return _gru_apply(x, partials, W_ih.T, W_hh.T,
                      b_ih.reshape(1, -1), b_hh.reshape(1, -1))
